# Initial kernel scaffold; baseline (speedup 1.0000x reference)
import jax
import jax.numpy as jnp
from jax.experimental import pallas as pl


def kernel(x, table):
    raise NotImplementedError("write your pallas kernel here")


# speedup vs baseline: 3.9504x; 3.9504x over previous
import functools

import jax
import jax.numpy as jnp
from jax import lax
from jax.experimental import pallas as pl
from jax.experimental.pallas import tpu as pltpu
from jax.experimental.pallas import tpu_sc as plsc

VOCAB = 100000
EMBED_DIM = 64
BATCH = 4096
HIST_LEN = 200

N = BATCH * HIST_LEN
NC, NS = 2, 16
NW = NC * NS
IDX_ROW = 128
ROWS_TOTAL = N // IDX_ROW
ROWS_PER_W = ROWS_TOTAL // NW
K = 4
CHUNK = K * IDX_ROW
N_CHUNKS = ROWS_PER_W // K


def _make_embed():
    mesh = plsc.VectorSubcoreMesh(core_axis_name="c", subcore_axis_name="s")

    @functools.partial(
        pl.kernel,
        mesh=mesh,
        compiler_params=pltpu.CompilerParams(use_tc_tiling_on_sc=False),
        out_type=jax.ShapeDtypeStruct((N, EMBED_DIM), jnp.float32),
        scratch_types=[
            pltpu.VMEM((K, IDX_ROW), jnp.int32),
            pltpu.VMEM((CHUNK, EMBED_DIM), jnp.float32),
            pltpu.SemaphoreType.DMA,
        ],
    )
    def embed(x_hbm, table_hbm, out_hbm, idx_v, rows_v, sem):
        wid = lax.axis_index("s") * NC + lax.axis_index("c")
        row_base = wid * ROWS_PER_W

        def chunk_body(i, carry):
            r0 = row_base + i * K
            pltpu.sync_copy(x_hbm.at[pl.ds(r0, K)], idx_v)
            cps = [
                pltpu.async_copy(
                    table_hbm.at[idx_v.at[j]],
                    rows_v.at[pl.ds(j * IDX_ROW, IDX_ROW)],
                    sem,
                )
                for j in range(K)
            ]
            for c in cps:
                c.wait()
            pltpu.sync_copy(rows_v, out_hbm.at[pl.ds(r0 * IDX_ROW, CHUNK)])
            return carry

        lax.fori_loop(0, N_CHUNKS, chunk_body, 0)

    return embed


_embed = _make_embed()


def kernel(x, table):
    x2 = x.reshape(ROWS_TOTAL, IDX_ROW).astype(jnp.int32)
    out = _embed(x2, table)
    return out.reshape(BATCH, HIST_LEN, EMBED_DIM)


# speedup vs baseline: 4.2655x; 1.0797x over previous
import functools

import jax
import jax.numpy as jnp
from jax import lax
from jax.experimental import pallas as pl
from jax.experimental.pallas import tpu as pltpu
from jax.experimental.pallas import tpu_sc as plsc

VOCAB = 100000
EMBED_DIM = 64
BATCH = 4096
HIST_LEN = 200

N = BATCH * HIST_LEN
NC, NS = 2, 16
NW = NC * NS
IDX_ROW = 128
ROWS_TOTAL = N // IDX_ROW
ROWS_PER_W = ROWS_TOTAL // NW
K = 4
CHUNK = K * IDX_ROW
N_CHUNKS = ROWS_PER_W // K


def _make_embed():
    mesh = plsc.VectorSubcoreMesh(core_axis_name="c", subcore_axis_name="s")

    @functools.partial(
        pl.kernel,
        mesh=mesh,
        compiler_params=pltpu.CompilerParams(use_tc_tiling_on_sc=False),
        out_type=jax.ShapeDtypeStruct((N, EMBED_DIM), jnp.float32),
        scratch_types=[
            pltpu.VMEM((K, IDX_ROW), jnp.int32),
            pltpu.VMEM((K, IDX_ROW), jnp.int32),
            pltpu.VMEM((CHUNK, EMBED_DIM), jnp.float32),
            pltpu.VMEM((CHUNK, EMBED_DIM), jnp.float32),
            pltpu.SemaphoreType.DMA,
            pltpu.SemaphoreType.DMA,
            pltpu.SemaphoreType.DMA,
            pltpu.SemaphoreType.DMA,
        ],
    )
    def embed(x_hbm, table_hbm, out_hbm, idx0, idx1, rows0, rows1,
              g0, g1, o0, o1):
        wid = lax.axis_index("s") * NC + lax.axis_index("c")
        row_base = wid * ROWS_PER_W
        idx_v = (idx0, idx1)
        rows_v = (rows0, rows1)
        gsem = (g0, g1)
        osem = (o0, o1)

        def load_idx(c, b):
            pltpu.sync_copy(x_hbm.at[pl.ds(row_base + c * K, K)], idx_v[b])

        def start_gather(b):
            for j in range(K):
                pltpu.async_copy(
                    table_hbm.at[idx_v[b].at[j]],
                    rows_v[b].at[pl.ds(j * IDX_ROW, IDX_ROW)],
                    gsem[b],
                )

        def wait_gather(b):
            for j in range(K):
                pltpu.make_async_copy(
                    table_hbm.at[idx_v[b].at[j]],
                    rows_v[b].at[pl.ds(j * IDX_ROW, IDX_ROW)],
                    gsem[b],
                ).wait()

        def start_write(c, b):
            pltpu.async_copy(
                rows_v[b],
                out_hbm.at[pl.ds((row_base + c * K) * IDX_ROW, CHUNK)],
                osem[b],
            )

        def wait_write(b):
            pltpu.make_async_copy(
                rows_v[b], out_hbm.at[pl.ds(0, CHUNK)], osem[b]
            ).wait()

        def step(c, b, do_wait_write, do_next_gather, do_prefetch):
            if do_next_gather:
                if do_wait_write:
                    wait_write(1 - b)
                start_gather(1 - b)
            wait_gather(b)
            start_write(c, b)
            if do_prefetch:
                load_idx(c + 2, b)

        load_idx(0, 0)
        start_gather(0)
        load_idx(1, 1)

        step(0, 0, False, True, True)
        step(1, 1, True, True, True)

        def interior(t, carry):
            step(2 * t, 0, True, True, True)
            step(2 * t + 1, 1, True, True, True)
            return carry

        lax.fori_loop(1, N_CHUNKS // 2 - 1, interior, 0)

        step(N_CHUNKS - 2, 0, True, True, False)
        step(N_CHUNKS - 1, 1, True, False, False)
        wait_write(0)
        wait_write(1)

    return embed


_embed = _make_embed()


def kernel(x, table):
    x2 = x.reshape(ROWS_TOTAL, IDX_ROW).astype(jnp.int32)
    out = _embed(x2, table)
    return out.reshape(BATCH, HIST_LEN, EMBED_DIM)
